# Initial kernel scaffold; baseline (speedup 1.0000x reference)
import jax
import jax.numpy as jnp
from jax.experimental import pallas as pl


def kernel(x, edge_index):
    raise NotImplementedError("write your pallas kernel here")


# speedup vs baseline: 3.1998x; 3.1998x over previous
import functools

import jax
import jax.numpy as jnp
from jax import lax
from jax.experimental import pallas as pl
from jax.experimental.pallas import tpu as pltpu
from jax.experimental.pallas import tpu_sc as plsc

N_NODES = 10000
N_EDGES = 320000
D_FEAT = 128

NUM_CORES = 2
NUM_SUBCORES = 16
NUM_WORKERS = NUM_CORES * NUM_SUBCORES

BLK = 128
NB = 80
EDGES_PER_WORKER = NB * BLK
E_PAD = NUM_WORKERS * EDGES_PER_WORKER
N_PAD = NUM_SUBCORES * 640
ZROWS = 640


def _sc_scatter(x_hbm, srcs_hbm, dsts_hbm, zeros_hbm, out_hbm,
                src_v, dst_v, rows_v, acc_sh, sem):
  c = lax.axis_index("c")
  s = lax.axis_index("s")
  w = c * NUM_SUBCORES + s

  pltpu.sync_copy(srcs_hbm.at[w], src_v)
  pltpu.sync_copy(dsts_hbm.at[w], dst_v)

  pltpu.sync_copy(zeros_hbm, acc_sh.at[pl.ds(s * ZROWS, ZROWS)])
  plsc.subcore_barrier()

  def step(b, carry):
    cp = pltpu.make_async_copy(x_hbm.at[src_v.at[b]], rows_v.at[0], sem)
    cp.start()
    cp.wait()
    pltpu.sync_copy(rows_v.at[0], acc_sh.at[dst_v.at[b]], add=True)
    return carry

  lax.fori_loop(0, NB, step, 0)

  plsc.subcore_barrier()
  pltpu.sync_copy(acc_sh.at[pl.ds(s * ZROWS, ZROWS)],
                  out_hbm.at[c].at[pl.ds(s * ZROWS, ZROWS)])


@functools.partial(
    pl.kernel,
    out_type=jax.ShapeDtypeStruct((NUM_CORES, N_PAD, D_FEAT), jnp.float32),
    mesh=plsc.VectorSubcoreMesh(core_axis_name="c", subcore_axis_name="s"),
    scratch_types=[
        pltpu.VMEM((NB, BLK), jnp.int32),
        pltpu.VMEM((NB, BLK), jnp.int32),
        pltpu.VMEM((1, BLK, D_FEAT), jnp.float32),
        pltpu.VMEM_SHARED((N_PAD, D_FEAT), jnp.float32),
        pltpu.SemaphoreType.DMA,
    ],
)
def _mp_scatter_kernel(x_hbm, srcs_hbm, dsts_hbm, zeros_hbm, out_hbm,
                       src_v, dst_v, rows_v, acc_sh, sem):
  _sc_scatter(x_hbm, srcs_hbm, dsts_hbm, zeros_hbm, out_hbm,
              src_v, dst_v, rows_v, acc_sh, sem)


def _combine_body(a_ref, b_ref, o_ref):
  o_ref[...] = a_ref[...] + b_ref[...]


def _combine(partials):
  blk = 1024
  out = pl.pallas_call(
      _combine_body,
      grid=(N_PAD // blk,),
      in_specs=[
          pl.BlockSpec((blk, D_FEAT), lambda i: (i, 0)),
          pl.BlockSpec((blk, D_FEAT), lambda i: (i, 0)),
      ],
      out_specs=pl.BlockSpec((blk, D_FEAT), lambda i: (i, 0)),
      out_shape=jax.ShapeDtypeStruct((N_PAD, D_FEAT), jnp.float32),
  )(partials[0], partials[1])
  return out[:N_NODES]


@jax.jit
def kernel(x, edge_index):
  src = edge_index[0].astype(jnp.int32)
  dst = edge_index[1].astype(jnp.int32)
  pad = E_PAD - N_EDGES
  src_p = jnp.concatenate([src, jnp.zeros((pad,), jnp.int32)])
  dst_p = jnp.concatenate([dst, jnp.full((pad,), N_NODES, jnp.int32)])
  srcs = src_p.reshape(NUM_WORKERS, NB, BLK)
  dsts = dst_p.reshape(NUM_WORKERS, NB, BLK)
  zeros = jnp.zeros((ZROWS, D_FEAT), jnp.float32)
  partials = _mp_scatter_kernel(x, srcs, dsts, zeros)
  return _combine(partials)


# speedup vs baseline: 3.5064x; 1.0958x over previous
import functools

import jax
import jax.numpy as jnp
from jax import lax
from jax.experimental import pallas as pl
from jax.experimental.pallas import tpu as pltpu
from jax.experimental.pallas import tpu_sc as plsc

N_NODES = 10000
N_EDGES = 320000
D_FEAT = 128

NUM_CORES = 2
NUM_SUBCORES = 16
NUM_WORKERS = NUM_CORES * NUM_SUBCORES

BLK = 128
NB = 80
EDGES_PER_WORKER = NB * BLK
E_PAD = NUM_WORKERS * EDGES_PER_WORKER
N_PAD = NUM_SUBCORES * 640
ZROWS = 640
NBUF = 2
NB_CHUNK = 40


def _sc_scatter(x_hbm, srcs_hbm, dsts_hbm, zeros_hbm, out_hbm,
                src_v, dst_v, rows_v, acc_sh, sem):
  c = lax.axis_index("c")
  s = lax.axis_index("s")
  w = c * NUM_SUBCORES + s

  pltpu.sync_copy(zeros_hbm, acc_sh.at[pl.ds(s * ZROWS, ZROWS)])
  plsc.subcore_barrier()

  def gather(b, k):
    return pltpu.make_async_copy(x_hbm.at[src_v.at[b]], rows_v.at[k],
                                 sem.at[k])

  for h in range(NB // NB_CHUNK):
    pltpu.sync_copy(srcs_hbm.at[w].at[pl.ds(h * NB_CHUNK, NB_CHUNK)], src_v)
    pltpu.sync_copy(dsts_hbm.at[w].at[pl.ds(h * NB_CHUNK, NB_CHUNK)], dst_v)

    for k in range(NBUF):
      gather(k, k).start()

    def step(i, carry):
      for k in range(NBUF):
        b = i * NBUF + k
        gather(b, k).wait()
        pltpu.sync_copy(rows_v.at[k], acc_sh.at[dst_v.at[b]], add=True)

        @pl.when(b + NBUF < NB_CHUNK)
        def _():
          gather(b + NBUF, k).start()
      return carry

    lax.fori_loop(0, NB_CHUNK // NBUF, step, 0)

  plsc.subcore_barrier()
  pltpu.sync_copy(acc_sh.at[pl.ds(s * ZROWS, ZROWS)],
                  out_hbm.at[c].at[pl.ds(s * ZROWS, ZROWS)])


@functools.partial(
    pl.kernel,
    out_type=jax.ShapeDtypeStruct((NUM_CORES, N_PAD, D_FEAT), jnp.float32),
    mesh=plsc.VectorSubcoreMesh(core_axis_name="c", subcore_axis_name="s"),
    scratch_types=[
        pltpu.VMEM((NB_CHUNK, BLK), jnp.int32),
        pltpu.VMEM((NB_CHUNK, BLK), jnp.int32),
        pltpu.VMEM((NBUF, BLK, D_FEAT), jnp.float32),
        pltpu.VMEM_SHARED((N_PAD, D_FEAT), jnp.float32),
        pltpu.SemaphoreType.DMA((NBUF,)),
    ],
)
def _mp_scatter_kernel(x_hbm, srcs_hbm, dsts_hbm, zeros_hbm, out_hbm,
                       src_v, dst_v, rows_v, acc_sh, sem):
  _sc_scatter(x_hbm, srcs_hbm, dsts_hbm, zeros_hbm, out_hbm,
              src_v, dst_v, rows_v, acc_sh, sem)


def _combine_body(a_ref, b_ref, o_ref):
  o_ref[...] = a_ref[...] + b_ref[...]


def _combine(partials):
  blk = 1024
  out = pl.pallas_call(
      _combine_body,
      grid=(N_PAD // blk,),
      in_specs=[
          pl.BlockSpec((blk, D_FEAT), lambda i: (i, 0)),
          pl.BlockSpec((blk, D_FEAT), lambda i: (i, 0)),
      ],
      out_specs=pl.BlockSpec((blk, D_FEAT), lambda i: (i, 0)),
      out_shape=jax.ShapeDtypeStruct((N_PAD, D_FEAT), jnp.float32),
  )(partials[0], partials[1])
  return out[:N_NODES]


@jax.jit
def kernel(x, edge_index):
  src = edge_index[0].astype(jnp.int32)
  dst = edge_index[1].astype(jnp.int32)
  pad = E_PAD - N_EDGES
  src_p = jnp.concatenate([src, jnp.zeros((pad,), jnp.int32)])
  dst_p = jnp.concatenate([dst, jnp.full((pad,), N_NODES, jnp.int32)])
  srcs = src_p.reshape(NUM_WORKERS, NB, BLK)
  dsts = dst_p.reshape(NUM_WORKERS, NB, BLK)
  zeros = jnp.zeros((ZROWS, D_FEAT), jnp.float32)
  partials = _mp_scatter_kernel(x, srcs, dsts, zeros)
  return _combine(partials)
